# speedup vs baseline: 144.4231x; 144.4231x over previous
import functools

import jax
import jax.numpy as jnp
from jax import lax
from jax.experimental import pallas as pl
from jax.experimental.pallas import tpu as pltpu
from jax.experimental.pallas import tpu_sc as plsc

_N = 16384 * 128
_NW = 32
_PER_W = _N // _NW
_CHUNK = 8192
_NCHUNK = _PER_W // _CHUNK
_L = 16


def _bsgen_body(src_hbm, seq_hbm, idx_hbm, out_hbm, idx_v, gat_v, src_v,
                out_v, sem):
    wid = lax.axis_index("s") * 2 + lax.axis_index("c")
    base = wid * _PER_W

    def chunk_body(c, _):
        off = base + c * _CHUNK
        pltpu.sync_copy(idx_hbm.at[pl.ds(off, _CHUNK)], idx_v)
        pltpu.async_copy(seq_hbm.at[idx_v], gat_v, sem).wait()
        pltpu.sync_copy(src_hbm.at[pl.ds(off, _CHUNK)], src_v)

        def cmp_body(i, _):
            s = src_v[pl.ds(i * _L, _L)]
            g = gat_v[pl.ds(i * _L, _L)]
            out_v[pl.ds(i * _L, _L)] = jnp.where(
                s > g, jnp.float32(1.0), jnp.float32(0.0))
            return 0

        lax.fori_loop(0, _CHUNK // _L, cmp_body, 0, unroll=4)
        pltpu.sync_copy(out_v, out_hbm.at[pl.ds(off, _CHUNK)])
        return 0

    lax.fori_loop(0, _NCHUNK, chunk_body, 0)


@jax.jit
def _bsgen(src, seq, idx):
    mesh = plsc.VectorSubcoreMesh(core_axis_name="c", subcore_axis_name="s")
    return pl.kernel(
        _bsgen_body,
        out_type=jax.ShapeDtypeStruct((_N,), jnp.float32),
        mesh=mesh,
        scratch_types=[
            pltpu.VMEM((_CHUNK,), jnp.int32),
            pltpu.VMEM((_CHUNK,), jnp.float32),
            pltpu.VMEM((_CHUNK,), jnp.float32),
            pltpu.VMEM((_CHUNK,), jnp.float32),
            pltpu.SemaphoreType.DMA,
        ],
    )(src, seq, idx)


def kernel(source, rng_seq, rng_idx):
    idx = rng_idx.astype(jnp.int32).reshape(_N)
    src = source.reshape(_N)
    out = _bsgen(src, rng_seq, idx)
    return out.reshape(source.shape)


# speedup vs baseline: 179.9010x; 1.2457x over previous
import jax
import jax.numpy as jnp
from jax import lax
from jax.experimental import pallas as pl
from jax.experimental.pallas import tpu as pltpu
from jax.experimental.pallas import tpu_sc as plsc

_N = 16384 * 128
_NW = 32
_PER_W = _N // _NW
_CHUNK = 8192
_NCHUNK = _PER_W // _CHUNK
_L = 16


def _bsgen_body(src_hbm, seq_hbm, idx_hbm, out_hbm, idx_all, gat0, gat1,
                src0, src1, out0, out1, sg0, sg1, ss0, ss1, so0, so1):
    wid = lax.axis_index("s") * 2 + lax.axis_index("c")
    base = wid * _PER_W

    gat = (gat0, gat1)
    src = (src0, src1)
    out = (out0, out1)
    sg = (sg0, sg1)
    ss = (ss0, ss1)
    so = (so0, so1)

    pltpu.sync_copy(idx_hbm.at[pl.ds(base, _PER_W)], idx_all)

    def fire(c):
        b = c & 1
        g = pltpu.async_copy(
            seq_hbm.at[idx_all.at[pl.ds(c * _CHUNK, _CHUNK)]], gat[b], sg[b])
        s = pltpu.async_copy(
            src_hbm.at[pl.ds(base + c * _CHUNK, _CHUNK)], src[b], ss[b])
        return g, s

    copies = {}
    out_copies = {}
    copies[0] = fire(0)
    copies[1] = fire(1)

    for c in range(_NCHUNK):
        b = c & 1
        g, s = copies.pop(c)
        g.wait()
        s.wait()
        if c - 2 >= 0:
            out_copies.pop(c - 2).wait()

        def cmp_body(i, _, b=b):
            sv = src[b][pl.ds(i * _L, _L)]
            gv = gat[b][pl.ds(i * _L, _L)]
            out[b][pl.ds(i * _L, _L)] = jnp.where(
                sv > gv, jnp.float32(1.0), jnp.float32(0.0))
            return 0

        lax.fori_loop(0, _CHUNK // _L, cmp_body, 0, unroll=8)
        out_copies[c] = pltpu.async_copy(
            out[b], out_hbm.at[pl.ds(base + c * _CHUNK, _CHUNK)], so[b])
        if c + 2 < _NCHUNK:
            copies[c + 2] = fire(c + 2)

    out_copies.pop(_NCHUNK - 2).wait()
    out_copies.pop(_NCHUNK - 1).wait()


@jax.jit
def _bsgen(src, seq, idx):
    mesh = plsc.VectorSubcoreMesh(core_axis_name="c", subcore_axis_name="s")
    return pl.kernel(
        _bsgen_body,
        out_type=jax.ShapeDtypeStruct((_N,), jnp.float32),
        mesh=mesh,
        scratch_types=[
            pltpu.VMEM((_PER_W,), jnp.int32),
            pltpu.VMEM((_CHUNK,), jnp.float32),
            pltpu.VMEM((_CHUNK,), jnp.float32),
            pltpu.VMEM((_CHUNK,), jnp.float32),
            pltpu.VMEM((_CHUNK,), jnp.float32),
            pltpu.VMEM((_CHUNK,), jnp.float32),
            pltpu.VMEM((_CHUNK,), jnp.float32),
            pltpu.SemaphoreType.DMA,
            pltpu.SemaphoreType.DMA,
            pltpu.SemaphoreType.DMA,
            pltpu.SemaphoreType.DMA,
            pltpu.SemaphoreType.DMA,
            pltpu.SemaphoreType.DMA,
        ],
    )(src, seq, idx)


def kernel(source, rng_seq, rng_idx):
    idx = rng_idx.astype(jnp.int32).reshape(_N)
    src = source.reshape(_N)
    out = _bsgen(src, rng_seq, idx)
    return out.reshape(source.shape)


# speedup vs baseline: 294.7777x; 1.6386x over previous
import jax
import jax.numpy as jnp
from jax import lax
from jax.experimental import pallas as pl
from jax.experimental.pallas import tpu as pltpu
from jax.experimental.pallas import tpu_sc as plsc

_N = 16384 * 128
_NW = 32
_PER_W = _N // _NW
_CHUNK = 4096
_NCHUNK = _PER_W // _CHUNK
_L = 16
_SEQ = 1000000
_SEQ_PART = 62496
_STAGE = 4096


def _bsgen_body(src_hbm, seq_hbm, idx_hbm, out_hbm, seq_sh, idx0, idx1, gat0,
                gat1, src0, src1, out0, out1, si0, si1, sg0, sg1, ss0, ss1,
                so0, so1):
    sid = lax.axis_index("s")
    wid = sid * 2 + lax.axis_index("c")
    base = wid * _PER_W

    idx = (idx0, idx1)
    gat = (gat0, gat1)
    src = (src0, src1)
    out = (out0, out1)
    si = (si0, si1)
    sg = (sg0, sg1)
    ss = (ss0, ss1)
    so = (so0, so1)

    sbase = sid * _SEQ_PART
    sizes = [_STAGE] * (_SEQ_PART // _STAGE) + [_SEQ_PART % _STAGE]
    cp_in = {0: pltpu.async_copy(
        seq_hbm.at[pl.ds(sbase, sizes[0])], gat[0].at[pl.ds(0, sizes[0])],
        sg[0])}
    for k in range(len(sizes)):
        b = k & 1
        if k + 1 < len(sizes):
            nb = (k + 1) & 1
            cp_in[k + 1] = pltpu.async_copy(
                seq_hbm.at[pl.ds(sbase + (k + 1) * _STAGE, sizes[k + 1])],
                gat[nb].at[pl.ds(0, sizes[k + 1])], sg[nb])
        cp_in.pop(k).wait()
        pltpu.sync_copy(gat[b].at[pl.ds(0, sizes[k])],
                        seq_sh.at[pl.ds(sbase + k * _STAGE, sizes[k])])

    @pl.when(sid == 15)
    def _copy_tail():
        tail = 16 * _SEQ_PART
        pltpu.sync_copy(seq_hbm.at[pl.ds(tail, _SEQ - tail)],
                        gat0.at[pl.ds(0, _SEQ - tail)])
        pltpu.sync_copy(gat0.at[pl.ds(0, _SEQ - tail)],
                        seq_sh.at[pl.ds(tail, _SEQ - tail)])

    plsc.subcore_barrier()

    def fire_idx(c):
        b = c & 1
        return pltpu.async_copy(
            idx_hbm.at[pl.ds(base + c * _CHUNK, _CHUNK)], idx[b], si[b])

    def fire_gather(c):
        b = c & 1
        g = pltpu.async_copy(seq_sh.at[idx[b]], gat[b], sg[b])
        s = pltpu.async_copy(
            src_hbm.at[pl.ds(base + c * _CHUNK, _CHUNK)], src[b], ss[b])
        return g, s

    icp = {0: fire_idx(0), 1: fire_idx(1)}
    icp.pop(0).wait()
    copies = {0: fire_gather(0)}
    out_copies = {}

    for c in range(_NCHUNK):
        b = c & 1
        if c + 1 < _NCHUNK:
            icp.pop(c + 1).wait()
            copies[c + 1] = fire_gather(c + 1)
        g, s = copies.pop(c)
        g.wait()
        s.wait()
        if c - 2 >= 0:
            out_copies.pop(c - 2).wait()

        def cmp_body(i, _, b=b):
            sv = src[b][pl.ds(i * _L, _L)]
            gv = gat[b][pl.ds(i * _L, _L)]
            out[b][pl.ds(i * _L, _L)] = jnp.where(
                sv > gv, jnp.float32(1.0), jnp.float32(0.0))
            return 0

        lax.fori_loop(0, _CHUNK // _L, cmp_body, 0, unroll=8)
        out_copies[c] = pltpu.async_copy(
            out[b], out_hbm.at[pl.ds(base + c * _CHUNK, _CHUNK)], so[b])
        if c + 2 < _NCHUNK:
            icp[c + 2] = fire_idx(c + 2)

    out_copies.pop(_NCHUNK - 2).wait()
    out_copies.pop(_NCHUNK - 1).wait()


@jax.jit
def _bsgen(src, seq, idx):
    mesh = plsc.VectorSubcoreMesh(core_axis_name="c", subcore_axis_name="s")
    return pl.kernel(
        _bsgen_body,
        out_type=jax.ShapeDtypeStruct((_N,), jnp.float32),
        mesh=mesh,
        scratch_types=[
            pltpu.VMEM_SHARED((_SEQ,), jnp.float32),
            pltpu.VMEM((_CHUNK,), jnp.int32),
            pltpu.VMEM((_CHUNK,), jnp.int32),
            pltpu.VMEM((_CHUNK,), jnp.float32),
            pltpu.VMEM((_CHUNK,), jnp.float32),
            pltpu.VMEM((_CHUNK,), jnp.float32),
            pltpu.VMEM((_CHUNK,), jnp.float32),
            pltpu.VMEM((_CHUNK,), jnp.float32),
            pltpu.VMEM((_CHUNK,), jnp.float32),
            pltpu.SemaphoreType.DMA,
            pltpu.SemaphoreType.DMA,
            pltpu.SemaphoreType.DMA,
            pltpu.SemaphoreType.DMA,
            pltpu.SemaphoreType.DMA,
            pltpu.SemaphoreType.DMA,
            pltpu.SemaphoreType.DMA,
            pltpu.SemaphoreType.DMA,
        ],
    )(src, seq, idx)


def kernel(source, rng_seq, rng_idx):
    idx = rng_idx.astype(jnp.int32).reshape(_N)
    src = source.reshape(_N)
    out = _bsgen(src, rng_seq, idx)
    return out.reshape(source.shape)


# speedup vs baseline: 380.0933x; 1.2894x over previous
import jax
import jax.numpy as jnp
from jax import lax
from jax.experimental import pallas as pl
from jax.experimental.pallas import tpu as pltpu
from jax.experimental.pallas import tpu_sc as plsc

_N = 16384 * 128
_NW = 32
_PER_W = _N // _NW
_CHUNK = 4096
_NCHUNK = _PER_W // _CHUNK
_L = 16
_SEQ = 1000000
_SEQ_PART = 62496
_STAGE = 4096


def _bsgen_body(src_hbm, seq_hbm, idx_hbm, out_hbm, seq_sh, idx0, idx1, gat0,
                gat1, src0, src1, out0, out1, si0, si1, sg0, sg1, ss0, ss1,
                so0, so1):
    sid = lax.axis_index("s")
    wid = sid * 2 + lax.axis_index("c")
    base = wid * _PER_W

    idx = (idx0, idx1)
    gat = (gat0, gat1)
    src = (src0, src1)
    out = (out0, out1)
    si = (si0, si1)
    sg = (sg0, sg1)
    ss = (ss0, ss1)
    so = (so0, so1)

    _PROBE_STAGING = True
    _PROBE_PIPELINE = True
    _PROBE_COMPUTE = True
    sbase = sid * _SEQ_PART
    sizes = [_STAGE] * (_SEQ_PART // _STAGE) + [_SEQ_PART % _STAGE]
    if _PROBE_STAGING:
        cp_in = {0: pltpu.async_copy(
            seq_hbm.at[pl.ds(sbase, sizes[0])], gat[0].at[pl.ds(0, sizes[0])],
            sg[0])}
        for k in range(len(sizes)):
            b = k & 1
            if k + 1 < len(sizes):
                nb = (k + 1) & 1
                cp_in[k + 1] = pltpu.async_copy(
                    seq_hbm.at[pl.ds(sbase + (k + 1) * _STAGE, sizes[k + 1])],
                    gat[nb].at[pl.ds(0, sizes[k + 1])], sg[nb])
            cp_in.pop(k).wait()
            pltpu.sync_copy(gat[b].at[pl.ds(0, sizes[k])],
                            seq_sh.at[pl.ds(sbase + k * _STAGE, sizes[k])])

        @pl.when(sid == 15)
        def _copy_tail():
            tail = 16 * _SEQ_PART
            pltpu.sync_copy(seq_hbm.at[pl.ds(tail, _SEQ - tail)],
                            gat0.at[pl.ds(0, _SEQ - tail)])
            pltpu.sync_copy(gat0.at[pl.ds(0, _SEQ - tail)],
                            seq_sh.at[pl.ds(tail, _SEQ - tail)])

    plsc.subcore_barrier()
    if not _PROBE_PIPELINE:
        return

    def fire_idx(c):
        b = c & 1
        return pltpu.async_copy(
            idx_hbm.at[pl.ds(base + c * _CHUNK, _CHUNK)], idx[b], si[b])

    def fire_gather(c):
        b = c & 1
        g = pltpu.async_copy(seq_sh.at[idx[b]], gat[b], sg[b])
        s = pltpu.async_copy(
            src_hbm.at[pl.ds(base + c * _CHUNK, _CHUNK)], src[b], ss[b])
        return g, s

    icp = {0: fire_idx(0), 1: fire_idx(1)}
    icp.pop(0).wait()
    copies = {0: fire_gather(0)}
    out_copies = {}

    for c in range(_NCHUNK):
        b = c & 1
        if c + 1 < _NCHUNK:
            icp.pop(c + 1).wait()
            copies[c + 1] = fire_gather(c + 1)
        g, s = copies.pop(c)
        g.wait()
        s.wait()
        if c - 2 >= 0:
            out_copies.pop(c - 2).wait()

        def cmp_body(i, b=b):
            sv = src[b][pl.ds(i, _L)]
            gv = gat[b][pl.ds(i, _L)]
            out[b][pl.ds(i, _L)] = jnp.where(
                sv > gv, jnp.float32(1.0), jnp.float32(0.0))

        if _PROBE_COMPUTE:
            plsc.parallel_loop(0, _CHUNK, _L, unroll=8)(cmp_body)
        out_copies[c] = pltpu.async_copy(
            out[b], out_hbm.at[pl.ds(base + c * _CHUNK, _CHUNK)], so[b])
        if c + 2 < _NCHUNK:
            icp[c + 2] = fire_idx(c + 2)

    out_copies.pop(_NCHUNK - 2).wait()
    out_copies.pop(_NCHUNK - 1).wait()


@jax.jit
def _bsgen(src, seq, idx):
    mesh = plsc.VectorSubcoreMesh(core_axis_name="c", subcore_axis_name="s")
    return pl.kernel(
        _bsgen_body,
        out_type=jax.ShapeDtypeStruct((_N,), jnp.float32),
        mesh=mesh,
        scratch_types=[
            pltpu.VMEM_SHARED((_SEQ,), jnp.float32),
            pltpu.VMEM((_CHUNK,), jnp.int32),
            pltpu.VMEM((_CHUNK,), jnp.int32),
            pltpu.VMEM((_CHUNK,), jnp.float32),
            pltpu.VMEM((_CHUNK,), jnp.float32),
            pltpu.VMEM((_CHUNK,), jnp.float32),
            pltpu.VMEM((_CHUNK,), jnp.float32),
            pltpu.VMEM((_CHUNK,), jnp.float32),
            pltpu.VMEM((_CHUNK,), jnp.float32),
            pltpu.SemaphoreType.DMA,
            pltpu.SemaphoreType.DMA,
            pltpu.SemaphoreType.DMA,
            pltpu.SemaphoreType.DMA,
            pltpu.SemaphoreType.DMA,
            pltpu.SemaphoreType.DMA,
            pltpu.SemaphoreType.DMA,
            pltpu.SemaphoreType.DMA,
        ],
    )(src, seq, idx)


def kernel(source, rng_seq, rng_idx):
    idx = rng_idx.astype(jnp.int32).reshape(_N)
    src = source.reshape(_N)
    out = _bsgen(src, rng_seq, idx)
    return out.reshape(source.shape)


# speedup vs baseline: 403.1441x; 1.0606x over previous
import jax
import jax.numpy as jnp
from jax import lax
from jax.experimental import pallas as pl
from jax.experimental.pallas import tpu as pltpu
from jax.experimental.pallas import tpu_sc as plsc

_N = 16384 * 128
_NW = 32
_PER_W = _N // _NW
_CHUNK = 8192
_NCHUNK = _PER_W // _CHUNK
_L = 16
_SEQ = 1000000
_SEQ_PART = 62496
_STAGE = 8192


def _bsgen_body(src_hbm, seq_hbm, idx_hbm, out_hbm, seq_sh, idx0, idx1, gat0,
                gat1, src0, src1, out0, out1, si0, si1, sg0, sg1, ss0, ss1,
                so0, so1, st0, st1, st2, st3):
    sid = lax.axis_index("s")
    wid = sid * 2 + lax.axis_index("c")
    base = wid * _PER_W

    idx = (idx0, idx1)
    gat = (gat0, gat1)
    src = (src0, src1)
    out = (out0, out1)
    si = (si0, si1)
    sg = (sg0, sg1)
    ss = (ss0, ss1)
    so = (so0, so1)
    st_in = (st0, st1)
    st_out = (st2, st3)

    def fire_idx(c):
        b = c & 1
        return pltpu.async_copy(
            idx_hbm.at[pl.ds(base + c * _CHUNK, _CHUNK)], idx[b], si[b])

    def fire_src(c):
        b = c & 1
        return pltpu.async_copy(
            src_hbm.at[pl.ds(base + c * _CHUNK, _CHUNK)], src[b], ss[b])

    icp = {0: fire_idx(0), 1: fire_idx(1)}
    scp = {0: fire_src(0), 1: fire_src(1)}

    sbase = sid * _SEQ_PART
    sizes = [_STAGE] * (_SEQ_PART // _STAGE) + [_SEQ_PART % _STAGE]
    nst = len(sizes)

    def fire_stage_in(k):
        b = k & 1
        return pltpu.async_copy(
            seq_hbm.at[pl.ds(sbase + k * _STAGE, sizes[k])],
            gat[b].at[pl.ds(0, sizes[k])], st_in[b])

    def fire_stage_out(k):
        b = k & 1
        return pltpu.async_copy(
            gat[b].at[pl.ds(0, sizes[k])],
            seq_sh.at[pl.ds(sbase + k * _STAGE, sizes[k])], st_out[b])

    sin = {0: fire_stage_in(0), 1: fire_stage_in(1)}
    sout = {}
    for k in range(nst):
        sin.pop(k).wait()
        sout[k] = fire_stage_out(k)
        if k + 2 < nst:
            sout.pop(k).wait()
            sin[k + 2] = fire_stage_in(k + 2)
    for k in range(max(0, nst - 2), nst):
        sout.pop(k).wait()

    @pl.when(sid == 15)
    def _copy_tail():
        tail = 16 * _SEQ_PART
        pltpu.sync_copy(seq_hbm.at[pl.ds(tail, _SEQ - tail)],
                        gat0.at[pl.ds(0, _SEQ - tail)])
        pltpu.sync_copy(gat0.at[pl.ds(0, _SEQ - tail)],
                        seq_sh.at[pl.ds(tail, _SEQ - tail)])

    plsc.subcore_barrier()

    def fire_gather(c):
        b = c & 1
        return pltpu.async_copy(seq_sh.at[idx[b]], gat[b], sg[b])

    icp.pop(0).wait()
    gcp = {0: fire_gather(0)}
    ocp = {}

    for c in range(_NCHUNK):
        b = c & 1
        if c + 1 < _NCHUNK:
            icp.pop(c + 1).wait()
            gcp[c + 1] = fire_gather(c + 1)
        gcp.pop(c).wait()
        scp.pop(c).wait()
        if c - 2 >= 0:
            ocp.pop(c - 2).wait()

        def cmp_body(i, b=b):
            sv = src[b][pl.ds(i, _L)]
            gv = gat[b][pl.ds(i, _L)]
            out[b][pl.ds(i, _L)] = jnp.where(
                sv > gv, jnp.float32(1.0), jnp.float32(0.0))

        plsc.parallel_loop(0, _CHUNK, _L, unroll=8)(cmp_body)
        ocp[c] = pltpu.async_copy(
            out[b], out_hbm.at[pl.ds(base + c * _CHUNK, _CHUNK)], so[b])
        if c + 2 < _NCHUNK:
            icp[c + 2] = fire_idx(c + 2)
            scp[c + 2] = fire_src(c + 2)

    ocp.pop(_NCHUNK - 2).wait()
    ocp.pop(_NCHUNK - 1).wait()


@jax.jit
def _bsgen(src, seq, idx):
    mesh = plsc.VectorSubcoreMesh(core_axis_name="c", subcore_axis_name="s")
    return pl.kernel(
        _bsgen_body,
        out_type=jax.ShapeDtypeStruct((_N,), jnp.float32),
        mesh=mesh,
        scratch_types=[
            pltpu.VMEM_SHARED((_SEQ,), jnp.float32),
            pltpu.VMEM((_CHUNK,), jnp.int32),
            pltpu.VMEM((_CHUNK,), jnp.int32),
            pltpu.VMEM((_CHUNK,), jnp.float32),
            pltpu.VMEM((_CHUNK,), jnp.float32),
            pltpu.VMEM((_CHUNK,), jnp.float32),
            pltpu.VMEM((_CHUNK,), jnp.float32),
            pltpu.VMEM((_CHUNK,), jnp.float32),
            pltpu.VMEM((_CHUNK,), jnp.float32),
            pltpu.SemaphoreType.DMA,
            pltpu.SemaphoreType.DMA,
            pltpu.SemaphoreType.DMA,
            pltpu.SemaphoreType.DMA,
            pltpu.SemaphoreType.DMA,
            pltpu.SemaphoreType.DMA,
            pltpu.SemaphoreType.DMA,
            pltpu.SemaphoreType.DMA,
            pltpu.SemaphoreType.DMA,
            pltpu.SemaphoreType.DMA,
            pltpu.SemaphoreType.DMA,
            pltpu.SemaphoreType.DMA,
        ],
    )(src, seq, idx)


def kernel(source, rng_seq, rng_idx):
    idx = rng_idx.astype(jnp.int32).reshape(_N)
    src = source.reshape(_N)
    out = _bsgen(src, rng_seq, idx)
    return out.reshape(source.shape)


# speedup vs baseline: 412.7976x; 1.0239x over previous
import jax
import jax.numpy as jnp
from jax import lax
from jax.experimental import pallas as pl
from jax.experimental.pallas import tpu as pltpu
from jax.experimental.pallas import tpu_sc as plsc

_N = 16384 * 128
_NW = 32
_PER_W = _N // _NW
_CHUNK = 8192
_NCHUNK = _PER_W // _CHUNK
_L = 16
_SEQ = 1000000
_SEQ_PART = 62496
_STAGE = 8192


def _bsgen_body(src_hbm, seq_hbm, idx_hbm, out_hbm, seq_sh, idx0, idx1, gat0,
                gat1, src0, src1, out0, out1, si0, si1, sg0, sg1, ss0, ss1,
                so0, so1, st0, st1, st2, st3):
    sid = lax.axis_index("s")
    wid = sid * 2 + lax.axis_index("c")
    base = wid * _PER_W

    idx = (idx0, idx1)
    gat = (gat0, gat1)
    src = (src0, src1)
    out = (out0, out1)
    si = (si0, si1)
    sg = (sg0, sg1)
    ss = (ss0, ss1)
    so = (so0, so1)
    st_in = (st0, st1)
    st_out = (st2, st3)

    def fire_idx(c):
        b = c & 1
        return pltpu.async_copy(
            idx_hbm.at[pl.ds(base + c * _CHUNK, _CHUNK)], idx[b], si[b])

    def fire_src(c):
        b = c & 1
        return pltpu.async_copy(
            src_hbm.at[pl.ds(base + c * _CHUNK, _CHUNK)], src[b], ss[b])

    icp = {0: fire_idx(0), 1: fire_idx(1)}
    scp = {0: fire_src(0), 1: fire_src(1)}

    sbase = sid * _SEQ_PART
    sizes = [_STAGE] * (_SEQ_PART // _STAGE) + [_SEQ_PART % _STAGE]
    nst = len(sizes)
    sbufs = (gat0, gat1, out0, out1)
    sisems = (st0, st1, st2, st3)
    sosems = (sg0, sg1, so0, so1)

    def fire_stage_in(k):
        b = k % 4
        return pltpu.async_copy(
            seq_hbm.at[pl.ds(sbase + k * _STAGE, sizes[k])],
            sbufs[b].at[pl.ds(0, sizes[k])], sisems[b])

    def fire_stage_out(k):
        b = k % 4
        return pltpu.async_copy(
            sbufs[b].at[pl.ds(0, sizes[k])],
            seq_sh.at[pl.ds(sbase + k * _STAGE, sizes[k])], sosems[b])

    sin = {k: fire_stage_in(k) for k in range(min(4, nst))}
    sout = {}
    for k in range(nst):
        sin.pop(k).wait()
        sout[k] = fire_stage_out(k)
        if k + 4 < nst:
            sout.pop(k).wait()
            sin[k + 4] = fire_stage_in(k + 4)
    for k in sorted(sout):
        sout.pop(k).wait()

    @pl.when(sid == 15)
    def _copy_tail():
        tail = 16 * _SEQ_PART
        pltpu.sync_copy(seq_hbm.at[pl.ds(tail, _SEQ - tail)],
                        gat0.at[pl.ds(0, _SEQ - tail)])
        pltpu.sync_copy(gat0.at[pl.ds(0, _SEQ - tail)],
                        seq_sh.at[pl.ds(tail, _SEQ - tail)])

    plsc.subcore_barrier()

    def fire_gather(c):
        b = c & 1
        return pltpu.async_copy(seq_sh.at[idx[b]], gat[b], sg[b])

    icp.pop(0).wait()
    gcp = {0: fire_gather(0)}
    ocp = {}

    for c in range(_NCHUNK):
        b = c & 1
        if c + 1 < _NCHUNK:
            icp.pop(c + 1).wait()
            gcp[c + 1] = fire_gather(c + 1)
        gcp.pop(c).wait()
        scp.pop(c).wait()
        if c - 2 >= 0:
            ocp.pop(c - 2).wait()

        def cmp_body(i, b=b):
            sv = src[b][pl.ds(i, _L)]
            gv = gat[b][pl.ds(i, _L)]
            out[b][pl.ds(i, _L)] = jnp.where(
                sv > gv, jnp.float32(1.0), jnp.float32(0.0))

        plsc.parallel_loop(0, _CHUNK, _L, unroll=8)(cmp_body)
        ocp[c] = pltpu.async_copy(
            out[b], out_hbm.at[pl.ds(base + c * _CHUNK, _CHUNK)], so[b])
        if c + 2 < _NCHUNK:
            icp[c + 2] = fire_idx(c + 2)
            scp[c + 2] = fire_src(c + 2)

    ocp.pop(_NCHUNK - 2).wait()
    ocp.pop(_NCHUNK - 1).wait()


@jax.jit
def _bsgen(src, seq, idx):
    mesh = plsc.VectorSubcoreMesh(core_axis_name="c", subcore_axis_name="s")
    return pl.kernel(
        _bsgen_body,
        out_type=jax.ShapeDtypeStruct((_N,), jnp.float32),
        mesh=mesh,
        scratch_types=[
            pltpu.VMEM_SHARED((_SEQ,), jnp.float32),
            pltpu.VMEM((_CHUNK,), jnp.int32),
            pltpu.VMEM((_CHUNK,), jnp.int32),
            pltpu.VMEM((_CHUNK,), jnp.float32),
            pltpu.VMEM((_CHUNK,), jnp.float32),
            pltpu.VMEM((_CHUNK,), jnp.float32),
            pltpu.VMEM((_CHUNK,), jnp.float32),
            pltpu.VMEM((_CHUNK,), jnp.float32),
            pltpu.VMEM((_CHUNK,), jnp.float32),
            pltpu.SemaphoreType.DMA,
            pltpu.SemaphoreType.DMA,
            pltpu.SemaphoreType.DMA,
            pltpu.SemaphoreType.DMA,
            pltpu.SemaphoreType.DMA,
            pltpu.SemaphoreType.DMA,
            pltpu.SemaphoreType.DMA,
            pltpu.SemaphoreType.DMA,
            pltpu.SemaphoreType.DMA,
            pltpu.SemaphoreType.DMA,
            pltpu.SemaphoreType.DMA,
            pltpu.SemaphoreType.DMA,
        ],
    )(src, seq, idx)


def kernel(source, rng_seq, rng_idx):
    idx = rng_idx.astype(jnp.int32).reshape(_N)
    src = source.reshape(_N)
    out = _bsgen(src, rng_seq, idx)
    return out.reshape(source.shape)
